# baseline (speedup 1.0000x reference)
import functools

import jax
import jax.numpy as jnp
from jax import lax
from jax.experimental import pallas as pl
from jax.experimental.pallas import tpu as pltpu
from jax.experimental.pallas import tpu_sc as plsc

_K_NN = 5
_L = 16


def _bcast16(x, dtype=jnp.int32):
    return jnp.full((_L,), x, dtype)


def _iota16():
    return lax.broadcasted_iota(jnp.int32, (_L,), 0)


def _lanemax16(v):
    for sh in (8, 4, 2, 1):
        v = jnp.maximum(v, _take16(v, _iota16() ^ sh))
    return v


def _lanemin16_i32(v):
    for sh in (8, 4, 2, 1):
        v = jnp.minimum(v, _take16(v, _iota16() ^ sh))
    return v


def _take16(v, idxvec):
    dnums = lax.GatherDimensionNumbers(
        offset_dims=(), collapsed_slice_dims=(0,), start_index_map=(0,))
    return lax.gather(v, idxvec[:, None], dnums, slice_sizes=(1,),
                      mode=lax.GatherScatterMode.PROMISE_IN_BOUNDS)


def _rsqrt16(x):
    i = lax.bitcast_convert_type(x, jnp.int32)
    y = lax.bitcast_convert_type(jnp.int32(0x5F3759DF) - lax.shift_right_logical(i, 1),
                                 jnp.float32)
    for _ in range(4):
        y = y * (1.5 - 0.5 * x * y * y)
    return y


def _make_sc_kernel(m, d, s, c):
    info = plsc.get_sparse_core_info()
    nw = info.num_cores * info.num_subcores
    per_w = m // nw
    dch = d // _L
    sgr = s // _L
    mesh = plsc.VectorSubcoreMesh(core_axis_name="c", subcore_axis_name="s")

    @functools.partial(
        pl.kernel,
        out_type=jax.ShapeDtypeStruct((m, d), jnp.float32),
        mesh=mesh,
        scratch_types=[
            pltpu.VMEM((per_w,), jnp.int32),
            pltpu.VMEM((per_w, d), jnp.float32),
            pltpu.VMEM((_L,), jnp.int32),
            pltpu.VMEM((_L, s), jnp.float32),
            pltpu.VMEM((s,), jnp.float32),
            pltpu.VMEM((s,), jnp.float32),
            pltpu.VMEM((s,), jnp.float32),
            pltpu.VMEM((2, d // 2), jnp.int32),
            pltpu.VMEM((2, d // 2), jnp.float32),
            pltpu.VMEM((d,), jnp.float32),
            pltpu.SemaphoreType.DMA,
        ],
    )
    def sc_nn(q_hbm, lab_hbm, queue2_hbm, qflat_hbm, out_hbm,
              lab_v, q_v, idx_v, chunk_v, dot_v, nrm_v, score_v,
              colidx_v, col_v, rep_v, sem):
        wid = lax.axis_index("s") * info.num_cores + lax.axis_index("c")
        base = wid * per_w
        pltpu.sync_copy(lab_hbm.at[pl.ds(base, per_w)], lab_v)
        pltpu.sync_copy(q_hbm.at[pl.ds(base, per_w)], q_v)
        iota = _iota16()
        neg_inf = jnp.float32(-jnp.inf)

        def one_query(i, _):
            lab_grp = lab_v[pl.ds((i // _L) * _L, _L)]
            c_vec = _take16(lab_grp, _bcast16(i % _L))

            def zero_grp(k, _):
                z = jnp.zeros((_L,), jnp.float32)
                dot_v[pl.ds(k * _L, _L)] = z
                nrm_v[pl.ds(k * _L, _L)] = z
                return 0

            lax.fori_loop(0, sgr, zero_grp, 0)

            def do_chunk(dc, _):
                idx_v[...] = c_vec * d + dc * _L + iota
                cp = pltpu.async_copy(queue2_hbm.at[idx_v], chunk_v, sem)
                q_grp = q_v[i, pl.ds(dc * _L, _L)]
                qds = [_take16(q_grp, _bcast16(dl)) for dl in range(_L)]
                cp.wait()

                def do_grp(k, _):
                    off = k * _L
                    acc_d = dot_v[pl.ds(off, _L)]
                    acc_n = nrm_v[pl.ds(off, _L)]
                    for dl in range(_L):
                        v = chunk_v[dl, pl.ds(off, _L)]
                        acc_d = acc_d + qds[dl] * v
                        acc_n = acc_n + v * v
                    dot_v[pl.ds(off, _L)] = acc_d
                    nrm_v[pl.ds(off, _L)] = acc_n
                    return 0

                lax.fori_loop(0, sgr, do_grp, 0)
                return 0

            lax.fori_loop(0, dch, do_chunk, 0)

            def do_score(k, _):
                sd = dot_v[pl.ds(k * _L, _L)]
                sn = nrm_v[pl.ds(k * _L, _L)]
                score_v[pl.ds(k * _L, _L)] = sd * jnp.abs(sd) / sn
                return 0

            lax.fori_loop(0, sgr, do_score, 0)

            kth = jnp.int32(0)
            for _r in range(_K_NN):
                def run_max(k, mv):
                    return jnp.maximum(mv, score_v[pl.ds(k * _L, _L)])

                mvec = lax.fori_loop(0, sgr, run_max,
                                     jnp.full((_L,), neg_inf, jnp.float32))
                hm_b = _lanemax16(mvec)

                def find_min_idx(k, gm):
                    v = score_v[pl.ds(k * _L, _L)]
                    cand = jnp.where(v >= hm_b, k * _L + iota,
                                     jnp.int32(2 ** 30))
                    return jnp.minimum(gm, cand)

                gmin = lax.fori_loop(0, sgr, find_min_idx,
                                     jnp.full((_L,), 2 ** 30, jnp.int32))
                kth_b = _lanemin16_i32(gmin)
                kth = kth_b[0]
                kc = kth // _L
                blk = score_v[pl.ds(kc * _L, _L)]
                score_v[pl.ds(kc * _L, _L)] = jnp.where(
                    iota == kth_b % _L, neg_inf, blk)

            nk_grp = nrm_v[pl.ds((kth // _L) * _L, _L)]
            nk = _take16(nk_grp, _bcast16(kth % _L))
            scale = _rsqrt16(nk)
            kth_b = _bcast16(kth)
            half = d // 2 // _L
            for cc in range(dch):
                j, pos = cc // half, (cc % half) * _L
                colidx_v[j, pl.ds(pos, _L)] = (
                    c_vec * (d * s) + (cc * _L + iota) * s + kth_b)
            for j in range(2):
                pltpu.async_copy(qflat_hbm.at[colidx_v.at[j]], col_v.at[j],
                                 sem).wait()
            for cc in range(dch):
                j, pos = cc // half, (cc % half) * _L
                rep_v[pl.ds(cc * _L, _L)] = col_v[j, pl.ds(pos, _L)] * scale
            pltpu.sync_copy(rep_v, out_hbm.at[base + i])
            return 0

        lax.fori_loop(0, per_w, one_query, 0)

    return sc_nn


def kernel(q, labels, queue):
    n, d = q.shape
    c, _, s = queue.shape
    m = n // 2
    lab = labels[:m].astype(jnp.int32)
    perm = jnp.argsort(lab)
    lab_s = lab[perm]
    q_s = q[perm]
    queue2 = queue.reshape(c * d, s)
    qflat = queue.reshape(c * d * s)
    rep_s = _make_sc_kernel(m, d, s, c)(q_s, lab_s, queue2, qflat)
    return q.at[perm].set(rep_s)


# speedup vs baseline: 2.3469x; 2.3469x over previous
import functools

import jax
import jax.numpy as jnp
from jax import lax
from jax.experimental import pallas as pl
from jax.experimental.pallas import tpu as pltpu
from jax.experimental.pallas import tpu_sc as plsc

_K_NN = 5
_W = 8
_L = 16


def _dist_body(lab_ref, q_ref, *rest):
    queue_refs = rest[:_W]
    dist_ref = rest[_W]
    dists = []
    for j in range(_W):
        tile = queue_refs[j][0]
        qv = q_ref[j, 0]
        dot = jax.lax.dot_general(
            qv, tile, (((1,), (0,)), ((), ())),
            precision=jax.lax.Precision.HIGHEST,
            preferred_element_type=jnp.float32)
        nrm = jnp.sum(tile * tile, axis=0, keepdims=True)
        inv = 1.0 / (jnp.sqrt(nrm) + 1e-12)
        dists.append(nrm * inv * inv - 2.0 * dot * inv)
    dist_ref[:, 0, 0] = jnp.concatenate(dists, axis=0)


def _tc_dist(lab_s, q_s, queue, m, d, s):
    rows = m // _W

    def q_map(i, lr):
        return (0, i, 0, 0)

    def queue_map(j):
        def f(i, lr):
            return (lr[j * rows + i], 0, 0)
        return f

    dist = pl.pallas_call(
        _dist_body,
        grid_spec=pltpu.PrefetchScalarGridSpec(
            num_scalar_prefetch=1,
            grid=(rows,),
            in_specs=[pl.BlockSpec((_W, 1, 1, d), q_map)] +
                     [pl.BlockSpec((1, d, s), queue_map(j)) for j in range(_W)],
            out_specs=pl.BlockSpec((_W, 1, 1, s), q_map),
        ),
        out_shape=jax.ShapeDtypeStruct((_W, rows, 1, s), jnp.float32),
    )(lab_s, q_s.reshape(_W, rows, 1, d), *([queue] * _W))
    return dist.reshape(m, s)


def _bcast16(x, dtype=jnp.int32):
    return jnp.full((_L,), x, dtype)


def _iota16():
    return lax.broadcasted_iota(jnp.int32, (_L,), 0)


def _take16(v, idxvec):
    dnums = lax.GatherDimensionNumbers(
        offset_dims=(), collapsed_slice_dims=(0,), start_index_map=(0,))
    return lax.gather(v, idxvec[:, None], dnums, slice_sizes=(1,),
                      mode=lax.GatherScatterMode.PROMISE_IN_BOUNDS)


def _lanemin16(v):
    for sh in (8, 4, 2, 1):
        v = jnp.minimum(v, _take16(v, _iota16() ^ sh))
    return v


def _lanesum16(v):
    for sh in (8, 4, 2, 1):
        v = v + _take16(v, _iota16() ^ sh)
    return v


def _rsqrt16(x):
    i = lax.bitcast_convert_type(x, jnp.int32)
    y = lax.bitcast_convert_type(
        jnp.int32(0x5F3759DF) - lax.shift_right_logical(i, 1), jnp.float32)
    for _ in range(4):
        y = y * (1.5 - 0.5 * x * y * y)
    return y


def _make_sc_select(m, d, s, c):
    info = plsc.get_sparse_core_info()
    nw = info.num_cores * info.num_subcores
    per_w = m // nw
    sgr = s // _L
    half = d // 2 // _L
    mesh = plsc.VectorSubcoreMesh(core_axis_name="c", subcore_axis_name="s")

    @functools.partial(
        pl.kernel,
        out_type=jax.ShapeDtypeStruct((m, d), jnp.float32),
        mesh=mesh,
        scratch_types=[
            pltpu.VMEM((per_w,), jnp.int32),
            pltpu.VMEM((per_w, s), jnp.float32),
            pltpu.VMEM((s,), jnp.float32),
            pltpu.VMEM((2, d // 2), jnp.int32),
            pltpu.VMEM((2, d // 2), jnp.float32),
            pltpu.VMEM((d,), jnp.float32),
            pltpu.SemaphoreType.DMA,
        ],
    )
    def sc_select(dist_hbm, lab_hbm, qflat_hbm, out_hbm,
                  lab_v, dist_v, row_v, colidx_v, col_v, rep_v, sem):
        wid = lax.axis_index("s") * info.num_cores + lax.axis_index("c")
        base = wid * per_w
        pltpu.sync_copy(lab_hbm.at[pl.ds(base, per_w)], lab_v)
        pltpu.sync_copy(dist_hbm.at[pl.ds(base, per_w)], dist_v)
        iota = _iota16()
        pos_inf = jnp.float32(jnp.inf)

        def one_query(i, _):
            lab_grp = lab_v[pl.ds((i // _L) * _L, _L)]
            c_vec = _take16(lab_grp, _bcast16(i % _L))

            def cp_row(k, _):
                row_v[pl.ds(k * _L, _L)] = dist_v[i, pl.ds(k * _L, _L)]
                return 0

            lax.fori_loop(0, sgr, cp_row, 0)

            kth = jnp.int32(0)
            for _r in range(_K_NN):
                def run_min(k, mv):
                    return jnp.minimum(mv, row_v[pl.ds(k * _L, _L)])

                mvec = lax.fori_loop(0, sgr, run_min,
                                     jnp.full((_L,), pos_inf, jnp.float32))
                lo_b = _lanemin16(mvec)

                def find_min_idx(k, gm):
                    v = row_v[pl.ds(k * _L, _L)]
                    cand = jnp.where(v <= lo_b, k * _L + iota,
                                     jnp.int32(2 ** 30))
                    return jnp.minimum(gm, cand)

                gmin = lax.fori_loop(0, sgr, find_min_idx,
                                     jnp.full((_L,), 2 ** 30, jnp.int32))
                kth_b = _lanemin16(gmin)
                kth = kth_b[0]
                kc = kth // _L
                blk = row_v[pl.ds(kc * _L, _L)]
                row_v[pl.ds(kc * _L, _L)] = jnp.where(
                    iota == kth_b % _L, pos_inf, blk)

            kth_b = _bcast16(kth)
            for cc in range(d // _L):
                j, pos = cc // half, (cc % half) * _L
                colidx_v[j, pl.ds(pos, _L)] = (
                    c_vec * (d * s) + (cc * _L + iota) * s + kth_b)
            for j in range(2):
                pltpu.async_copy(qflat_hbm.at[colidx_v.at[j]], col_v.at[j],
                                 sem).wait()

            acc = jnp.zeros((_L,), jnp.float32)
            for j in range(2):
                for g in range(half):
                    v = col_v[j, pl.ds(g * _L, _L)]
                    acc = acc + v * v
            nrm2 = _lanesum16(acc)
            scale = _rsqrt16(jnp.maximum(nrm2, jnp.float32(1e-30)))
            for cc in range(d // _L):
                j, pos = cc // half, (cc % half) * _L
                rep_v[pl.ds(cc * _L, _L)] = col_v[j, pl.ds(pos, _L)] * scale
            pltpu.sync_copy(rep_v, out_hbm.at[base + i])
            return 0

        lax.fori_loop(0, per_w, one_query, 0)

    return sc_select


def kernel(q, labels, queue):
    n, d = q.shape
    c, _, s = queue.shape
    m = n // 2
    lab = labels[:m].astype(jnp.int32)
    perm = jnp.argsort(lab)
    lab_s = lab[perm]
    q_s = q[perm]
    dist = _tc_dist(lab_s, q_s, queue, m, d, s)
    qflat = queue.reshape(c * d * s)
    rep_s = _make_sc_select(m, d, s, c)(dist, lab_s, qflat)
    return q.at[perm].set(rep_s)


# speedup vs baseline: 2.6566x; 1.1320x over previous
import functools

import jax
import jax.numpy as jnp
from jax import lax
from jax.experimental import pallas as pl
from jax.experimental.pallas import tpu as pltpu
from jax.experimental.pallas import tpu_sc as plsc

_K_NN = 5
_W = 8
_L = 16
_LW = 128


def _dist_body(lab_ref, q_ref, *rest):
    queue_refs = rest[:_W]
    dist_ref = rest[_W]
    dists = []
    for j in range(_W):
        tile = queue_refs[j][0]
        qv = q_ref[j, 0]
        dot = jax.lax.dot_general(
            qv, tile, (((1,), (0,)), ((), ())),
            precision=jax.lax.Precision.HIGHEST,
            preferred_element_type=jnp.float32)
        nrm = jnp.sum(tile * tile, axis=0, keepdims=True)
        inv = 1.0 / (jnp.sqrt(nrm) + 1e-12)
        dists.append(nrm * inv * inv - 2.0 * dot * inv)
    dist_ref[:, 0, 0] = jnp.concatenate(dists, axis=0)


def _tc_dist(lab_s, q_s, queue, m, d, s):
    rows = m // _W

    def q_map(i, lr):
        return (0, i, 0, 0)

    def queue_map(j):
        def f(i, lr):
            return (lr[j * rows + i], 0, 0)
        return f

    dist = pl.pallas_call(
        _dist_body,
        grid_spec=pltpu.PrefetchScalarGridSpec(
            num_scalar_prefetch=1,
            grid=(rows,),
            in_specs=[pl.BlockSpec((_W, 1, 1, d), q_map)] +
                     [pl.BlockSpec((1, d, s), queue_map(j)) for j in range(_W)],
            out_specs=pl.BlockSpec((_W, 1, 1, s), q_map),
        ),
        out_shape=jax.ShapeDtypeStruct((_W, rows, 1, s), jnp.float32),
    )(lab_s, q_s.reshape(_W, rows, 1, d), *([queue] * _W))
    return dist.reshape(m, s)


def _bcast16(x, dtype=jnp.int32):
    return jnp.full((_L,), x, dtype)


def _iota16():
    return lax.broadcasted_iota(jnp.int32, (_L,), 0)


def _take16(v, idxvec):
    dnums = lax.GatherDimensionNumbers(
        offset_dims=(), collapsed_slice_dims=(0,), start_index_map=(0,))
    return lax.gather(v, idxvec[:, None], dnums, slice_sizes=(1,),
                      mode=lax.GatherScatterMode.PROMISE_IN_BOUNDS)


def _lanemin16(v):
    for sh in (8, 4, 2, 1):
        v = jnp.minimum(v, _take16(v, _iota16() ^ sh))
    return v


def _make_sc_select(m, s):
    info = plsc.get_sparse_core_info()
    nw = info.num_cores * info.num_subcores
    per_w = m // nw
    sgr = s // _L
    mesh = plsc.VectorSubcoreMesh(core_axis_name="c", subcore_axis_name="s")

    @functools.partial(
        pl.kernel,
        out_type=jax.ShapeDtypeStruct((m,), jnp.int32),
        mesh=mesh,
        scratch_types=[
            pltpu.VMEM((per_w, s), jnp.float32),
            pltpu.VMEM((s,), jnp.float32),
            pltpu.VMEM((per_w,), jnp.int32),
        ],
    )
    def sc_select(dist_hbm, out_hbm, dist_v, row_v, kth_v):
        wid = lax.axis_index("s") * info.num_cores + lax.axis_index("c")
        base = wid * per_w
        pltpu.sync_copy(dist_hbm.at[pl.ds(base, per_w)], dist_v)
        iota = _iota16()
        pos_inf = jnp.float32(jnp.inf)

        def one_group(g, _):
            def one_query(l, acc):
                i = g * _L + l

                def cp_row(k, _):
                    row_v[pl.ds(k * _L, _L)] = dist_v[i, pl.ds(k * _L, _L)]
                    return 0

                lax.fori_loop(0, sgr, cp_row, 0)

                kth_b = _bcast16(0)
                for _r in range(_K_NN):
                    def run_min(k, mv):
                        return jnp.minimum(mv, row_v[pl.ds(k * _L, _L)])

                    mvec = lax.fori_loop(0, sgr, run_min,
                                         jnp.full((_L,), pos_inf, jnp.float32))
                    lo_b = _lanemin16(mvec)

                    def find_min_idx(k, gm):
                        v = row_v[pl.ds(k * _L, _L)]
                        cand = jnp.where(v <= lo_b, k * _L + iota,
                                         jnp.int32(2 ** 30))
                        return jnp.minimum(gm, cand)

                    gmin = lax.fori_loop(0, sgr, find_min_idx,
                                         jnp.full((_L,), 2 ** 30, jnp.int32))
                    kth_b = _lanemin16(gmin)
                    kc = kth_b[0] // _L
                    blk = row_v[pl.ds(kc * _L, _L)]
                    row_v[pl.ds(kc * _L, _L)] = jnp.where(
                        iota == kth_b % _L, pos_inf, blk)

                return jnp.where(iota == l, kth_b, acc)

            picks = lax.fori_loop(0, _L, one_query, jnp.zeros((_L,), jnp.int32))
            kth_v[pl.ds(g * _L, _L)] = picks
            return 0

        lax.fori_loop(0, per_w // _L, one_group, 0)
        pltpu.sync_copy(kth_v, out_hbm.at[pl.ds(base, per_w)])

    return sc_select


def _extract_body(scal_ref, *rest):
    queue_refs = rest[:_W]
    out_ref = rest[_W]
    m = scal_ref.shape[0] // 2
    rows = m // _W
    i = pl.program_id(0)
    lane = jax.lax.broadcasted_iota(jnp.int32, (1, _LW), 1)
    for j in range(_W):
        win = queue_refs[j][0]
        kth = scal_ref[m + j * rows + i]
        mask = (lane == kth % _LW).astype(jnp.float32)
        col = jax.lax.dot_general(
            mask, win, (((1,), (1,)), ((), ())),
            preferred_element_type=jnp.float32)
        nrm = jnp.sum(col * col)
        out_ref[j, 0] = col / (jnp.sqrt(nrm) + 1e-12)


def _tc_extract(lab_ks, queue, m, d, s):
    rows = m // _W

    def out_map(i, sc):
        return (0, i, 0, 0)

    def queue_map(j):
        def f(i, sc):
            return (sc[j * rows + i], 0, sc[m + j * rows + i] // _LW)
        return f

    rep = pl.pallas_call(
        _extract_body,
        grid_spec=pltpu.PrefetchScalarGridSpec(
            num_scalar_prefetch=1,
            grid=(rows,),
            in_specs=[pl.BlockSpec((1, d, _LW), queue_map(j))
                      for j in range(_W)],
            out_specs=pl.BlockSpec((_W, 1, 1, d), out_map),
        ),
        out_shape=jax.ShapeDtypeStruct((_W, rows, 1, d), jnp.float32),
    )(lab_ks, *([queue] * _W))
    return rep.reshape(m, d)


def kernel(q, labels, queue):
    n, d = q.shape
    c, _, s = queue.shape
    m = n // 2
    lab = labels[:m].astype(jnp.int32)
    perm = jnp.argsort(lab)
    lab_s = lab[perm]
    q_s = q[perm]
    dist = _tc_dist(lab_s, q_s, queue, m, d, s)
    ks = _make_sc_select(m, s)(dist)
    rep_s = _tc_extract(jnp.concatenate([lab_s, ks]), queue, m, d, s)
    return q.at[perm].set(rep_s)


# speedup vs baseline: 2.7374x; 1.0304x over previous
import functools

import jax
import jax.numpy as jnp
from jax import lax
from jax.experimental import pallas as pl
from jax.experimental.pallas import tpu as pltpu
from jax.experimental.pallas import tpu_sc as plsc

_K_NN = 5
_W = 8
_L = 16
_LW = 128


def _dist_body(scal_ref, *rest):
    q_refs = rest[:_W]
    queue_refs = rest[_W:2 * _W]
    dist_ref = rest[2 * _W]
    dists = []
    for j in range(_W):
        tile = queue_refs[j][0]
        qv = q_refs[j][0]
        dot = jax.lax.dot_general(
            qv, tile, (((1,), (0,)), ((), ())),
            precision=jax.lax.Precision.HIGHEST,
            preferred_element_type=jnp.float32)
        nrm = jnp.sum(tile * tile, axis=0, keepdims=True)
        inv = 1.0 / (jnp.sqrt(nrm) + 1e-12)
        dists.append(nrm * inv * inv - 2.0 * dot * inv)
    dist_ref[:, 0, 0] = jnp.concatenate(dists, axis=0)


def _tc_dist(lab_perm, q, queue, m, d, s):
    rows = m // _W

    def out_map(i, sc):
        return (0, i, 0, 0)

    def q_map(j):
        def f(i, sc):
            return (sc[m + j * rows + i], 0, 0)
        return f

    def queue_map(j):
        def f(i, sc):
            return (sc[j * rows + i], 0, 0)
        return f

    dist = pl.pallas_call(
        _dist_body,
        grid_spec=pltpu.PrefetchScalarGridSpec(
            num_scalar_prefetch=1,
            grid=(rows,),
            in_specs=[pl.BlockSpec((1, 1, d), q_map(j)) for j in range(_W)] +
                     [pl.BlockSpec((1, d, s), queue_map(j)) for j in range(_W)],
            out_specs=pl.BlockSpec((_W, 1, 1, s), out_map),
        ),
        out_shape=jax.ShapeDtypeStruct((_W, rows, 1, s), jnp.float32),
    )(lab_perm, *([q.reshape(-1, 1, d)] * _W), *([queue] * _W))
    return dist.reshape(m, s)


def _bcast16(x, dtype=jnp.int32):
    return jnp.full((_L,), x, dtype)


def _iota16():
    return lax.broadcasted_iota(jnp.int32, (_L,), 0)


def _take16(v, idxvec):
    dnums = lax.GatherDimensionNumbers(
        offset_dims=(), collapsed_slice_dims=(0,), start_index_map=(0,))
    return lax.gather(v, idxvec[:, None], dnums, slice_sizes=(1,),
                      mode=lax.GatherScatterMode.PROMISE_IN_BOUNDS)


def _lanemin16(v):
    for sh in (8, 4, 2, 1):
        v = jnp.minimum(v, _take16(v, _iota16() ^ sh))
    return v


def _make_sc_select(m, s):
    info = plsc.get_sparse_core_info()
    nw = info.num_cores * info.num_subcores
    per_w = m // nw
    sgr = s // _L
    mesh = plsc.VectorSubcoreMesh(core_axis_name="c", subcore_axis_name="s")

    @functools.partial(
        pl.kernel,
        out_type=jax.ShapeDtypeStruct((m,), jnp.int32),
        mesh=mesh,
        scratch_types=[
            pltpu.VMEM((per_w, s), jnp.float32),
            pltpu.VMEM((s,), jnp.float32),
            pltpu.VMEM((per_w,), jnp.int32),
        ],
    )
    def sc_select(dist_hbm, out_hbm, dist_v, row_v, kth_v):
        wid = lax.axis_index("s") * info.num_cores + lax.axis_index("c")
        base = wid * per_w
        pltpu.sync_copy(dist_hbm.at[pl.ds(base, per_w)], dist_v)
        iota = _iota16()
        pos_inf = jnp.float32(jnp.inf)

        def one_group(g, _):
            def one_query(l, acc):
                i = g * _L + l

                def cp_row(k, _):
                    row_v[pl.ds(k * _L, _L)] = dist_v[i, pl.ds(k * _L, _L)]
                    return 0

                lax.fori_loop(0, sgr, cp_row, 0)

                kth_b = _bcast16(0)
                for _r in range(_K_NN):
                    def run_min(k, mv):
                        return jnp.minimum(mv, row_v[pl.ds(k * _L, _L)])

                    mvec = lax.fori_loop(0, sgr, run_min,
                                         jnp.full((_L,), pos_inf, jnp.float32))
                    lo_b = _lanemin16(mvec)

                    def find_min_idx(k, gm):
                        v = row_v[pl.ds(k * _L, _L)]
                        cand = jnp.where(v <= lo_b, k * _L + iota,
                                         jnp.int32(2 ** 30))
                        return jnp.minimum(gm, cand)

                    gmin = lax.fori_loop(0, sgr, find_min_idx,
                                         jnp.full((_L,), 2 ** 30, jnp.int32))
                    kth_b = _lanemin16(gmin)
                    kc = kth_b[0] // _L
                    blk = row_v[pl.ds(kc * _L, _L)]
                    row_v[pl.ds(kc * _L, _L)] = jnp.where(
                        iota == kth_b % _L, pos_inf, blk)

                return jnp.where(iota == l, kth_b, acc)

            picks = lax.fori_loop(0, _L, one_query, jnp.zeros((_L,), jnp.int32))
            kth_v[pl.ds(g * _L, _L)] = picks
            return 0

        lax.fori_loop(0, per_w // _L, one_group, 0)
        pltpu.sync_copy(kth_v, out_hbm.at[pl.ds(base, per_w)])

    return sc_select


def _extract_body(scal_ref, *rest):
    queue_refs = rest[:_W]
    out_ref = rest[_W]
    m = scal_ref.shape[0] // 2
    rows = m // _W
    i = pl.program_id(0)
    lane = jax.lax.broadcasted_iota(jnp.int32, (1, _LW), 1)
    for j in range(_W):
        win = queue_refs[j][0]
        kth = scal_ref[m + j * rows + i]
        mask = (lane == kth % _LW).astype(jnp.float32)
        col = jax.lax.dot_general(
            mask, win, (((1,), (1,)), ((), ())),
            preferred_element_type=jnp.float32)
        nrm = jnp.sum(col * col)
        out_ref[j, 0] = col / (jnp.sqrt(nrm) + 1e-12)


def _tc_extract(lab_ks, queue, m, d, s):
    rows = m // _W

    def out_map(i, sc):
        return (0, i, 0, 0)

    def queue_map(j):
        def f(i, sc):
            return (sc[j * rows + i], 0, sc[m + j * rows + i] // _LW)
        return f

    rep = pl.pallas_call(
        _extract_body,
        grid_spec=pltpu.PrefetchScalarGridSpec(
            num_scalar_prefetch=1,
            grid=(rows,),
            in_specs=[pl.BlockSpec((1, d, _LW), queue_map(j))
                      for j in range(_W)],
            out_specs=pl.BlockSpec((_W, 1, 1, d), out_map),
        ),
        out_shape=jax.ShapeDtypeStruct((_W, rows, 1, d), jnp.float32),
    )(lab_ks, *([queue] * _W))
    return rep.reshape(m, d)


def kernel(q, labels, queue):
    n, d = q.shape
    c, _, s = queue.shape
    m = n // 2
    lab = labels[:m].astype(jnp.int32)
    perm = jnp.argsort(lab)
    lab_s = lab[perm]
    mh = m // 2
    sel = _make_sc_select(mh, s)
    dist1 = _tc_dist(jnp.concatenate([lab_s[:mh], perm[:mh]]), q, queue,
                     mh, d, s)
    ks1 = sel(dist1)
    dist2 = _tc_dist(jnp.concatenate([lab_s[mh:], perm[mh:]]), q, queue,
                     mh, d, s)
    ks2 = sel(dist2)
    rep1 = _tc_extract(jnp.concatenate([lab_s[:mh], ks1]), queue, mh, d, s)
    rep2 = _tc_extract(jnp.concatenate([lab_s[mh:], ks2]), queue, mh, d, s)
    return q.at[perm].set(jnp.concatenate([rep1, rep2]))


# speedup vs baseline: 2.7673x; 1.0109x over previous
import functools

import jax
import jax.numpy as jnp
from jax import lax
from jax.experimental import pallas as pl
from jax.experimental.pallas import tpu as pltpu
from jax.experimental.pallas import tpu_sc as plsc

_K_NN = 5
_W = 8
_L = 16
_LW = 128


def _dist_body(scal_ref, *rest):
    q_refs = rest[:_W]
    queue_refs = rest[_W:2 * _W]
    dist_ref = rest[2 * _W]
    dists = []
    for j in range(_W):
        tile = queue_refs[j][0]
        qv = q_refs[j][0]
        dot = jax.lax.dot_general(
            qv, tile, (((1,), (0,)), ((), ())),
            precision=jax.lax.Precision.HIGHEST,
            preferred_element_type=jnp.float32)
        nrm = jnp.sum(tile * tile, axis=0, keepdims=True)
        inv = 1.0 / (jnp.sqrt(nrm) + 1e-12)
        dists.append(nrm * inv * inv - 2.0 * dot * inv)
    dist_ref[:, 0, 0] = jnp.concatenate(dists, axis=0)


def _tc_dist(lab_perm, q, queue, m, d, s):
    rows = m // _W

    def out_map(i, sc):
        return (0, i, 0, 0)

    def q_map(j):
        def f(i, sc):
            return (sc[m + j * rows + i], 0, 0)
        return f

    def queue_map(j):
        def f(i, sc):
            return (sc[j * rows + i], 0, 0)
        return f

    dist = pl.pallas_call(
        _dist_body,
        grid_spec=pltpu.PrefetchScalarGridSpec(
            num_scalar_prefetch=1,
            grid=(rows,),
            in_specs=[pl.BlockSpec((1, 1, d), q_map(j)) for j in range(_W)] +
                     [pl.BlockSpec((1, d, s), queue_map(j)) for j in range(_W)],
            out_specs=pl.BlockSpec((_W, 1, 1, s), out_map),
        ),
        out_shape=jax.ShapeDtypeStruct((_W, rows, 1, s), jnp.float32),
    )(lab_perm, *([q.reshape(-1, 1, d)] * _W), *([queue] * _W))
    return dist.reshape(m, s)


def _bcast16(x, dtype=jnp.int32):
    return jnp.full((_L,), x, dtype)


def _iota16():
    return lax.broadcasted_iota(jnp.int32, (_L,), 0)


def _take16(v, idxvec):
    dnums = lax.GatherDimensionNumbers(
        offset_dims=(), collapsed_slice_dims=(0,), start_index_map=(0,))
    return lax.gather(v, idxvec[:, None], dnums, slice_sizes=(1,),
                      mode=lax.GatherScatterMode.PROMISE_IN_BOUNDS)


def _lanemin16(v):
    for sh in (8, 4, 2, 1):
        v = jnp.minimum(v, _take16(v, _iota16() ^ sh))
    return v


def _make_sc_select(m, s):
    info = plsc.get_sparse_core_info()
    nw = info.num_cores * info.num_subcores
    per_w = m // nw
    sgr = s // _L
    mesh = plsc.VectorSubcoreMesh(core_axis_name="c", subcore_axis_name="s")

    @functools.partial(
        pl.kernel,
        out_type=jax.ShapeDtypeStruct((m,), jnp.int32),
        mesh=mesh,
        scratch_types=[
            pltpu.VMEM((per_w, s), jnp.float32),
            pltpu.VMEM((s,), jnp.float32),
            pltpu.VMEM((per_w,), jnp.int32),
        ],
    )
    def sc_select(dist_hbm, out_hbm, dist_v, row_v, kth_v):
        wid = lax.axis_index("s") * info.num_cores + lax.axis_index("c")
        base = wid * per_w
        pltpu.sync_copy(dist_hbm.at[pl.ds(base, per_w)], dist_v)
        iota = _iota16()
        pos_inf = jnp.float32(jnp.inf)

        def one_group(g, _):
            def one_query(l, acc):
                i = g * _L + l

                def cp_row(k, _):
                    row_v[pl.ds(k * _L, _L)] = dist_v[i, pl.ds(k * _L, _L)]
                    return 0

                lax.fori_loop(0, sgr, cp_row, 0)

                kth_b = _bcast16(0)
                for _r in range(_K_NN):
                    def run_min(k, mv):
                        return jnp.minimum(mv, row_v[pl.ds(k * _L, _L)])

                    mvec = lax.fori_loop(0, sgr, run_min,
                                         jnp.full((_L,), pos_inf, jnp.float32))
                    lo_b = _lanemin16(mvec)

                    def find_min_idx(k, gm):
                        v = row_v[pl.ds(k * _L, _L)]
                        cand = jnp.where(v <= lo_b, k * _L + iota,
                                         jnp.int32(2 ** 30))
                        return jnp.minimum(gm, cand)

                    gmin = lax.fori_loop(0, sgr, find_min_idx,
                                         jnp.full((_L,), 2 ** 30, jnp.int32))
                    kth_b = _lanemin16(gmin)
                    kc = kth_b[0] // _L
                    blk = row_v[pl.ds(kc * _L, _L)]
                    row_v[pl.ds(kc * _L, _L)] = jnp.where(
                        iota == kth_b % _L, pos_inf, blk)

                return jnp.where(iota == l, kth_b, acc)

            picks = lax.fori_loop(0, _L, one_query, jnp.zeros((_L,), jnp.int32))
            kth_v[pl.ds(g * _L, _L)] = picks
            return 0

        lax.fori_loop(0, per_w // _L, one_group, 0)
        pltpu.sync_copy(kth_v, out_hbm.at[pl.ds(base, per_w)])

    return sc_select


def _extract_body(scal_ref, *rest):
    queue_refs = rest[:_W]
    out_ref = rest[_W]
    m = scal_ref.shape[0] // 2
    rows = m // _W
    i = pl.program_id(0)
    lane = jax.lax.broadcasted_iota(jnp.int32, (1, _LW), 1)
    for j in range(_W):
        win = queue_refs[j][0]
        kth = scal_ref[m + j * rows + i]
        mask = (lane == kth % _LW).astype(jnp.float32)
        col = jax.lax.dot_general(
            mask, win, (((1,), (1,)), ((), ())),
            preferred_element_type=jnp.float32)
        nrm = jnp.sum(col * col)
        out_ref[j, 0] = col / (jnp.sqrt(nrm) + 1e-12)


def _tc_extract(lab_ks, queue, m, d, s):
    rows = m // _W

    def out_map(i, sc):
        return (0, i, 0, 0)

    def queue_map(j):
        def f(i, sc):
            return (sc[j * rows + i], 0, sc[m + j * rows + i] // _LW)
        return f

    rep = pl.pallas_call(
        _extract_body,
        grid_spec=pltpu.PrefetchScalarGridSpec(
            num_scalar_prefetch=1,
            grid=(rows,),
            in_specs=[pl.BlockSpec((1, d, _LW), queue_map(j))
                      for j in range(_W)],
            out_specs=pl.BlockSpec((_W, 1, 1, d), out_map),
        ),
        out_shape=jax.ShapeDtypeStruct((_W, rows, 1, d), jnp.float32),
    )(lab_ks, *([queue] * _W))
    return rep.reshape(m, d)


def kernel(q, labels, queue):
    n, d = q.shape
    c, _, s = queue.shape
    m = n // 2
    lab = labels[:m].astype(jnp.int32)
    packed = jnp.sort(lab * 2048 + jnp.arange(m, dtype=jnp.int32))
    perm = packed & 2047
    lab_s = packed >> 11
    mh = m // 2
    sel = _make_sc_select(mh, s)
    dist1 = _tc_dist(jnp.concatenate([lab_s[:mh], perm[:mh]]), q, queue,
                     mh, d, s)
    ks1 = sel(dist1)
    dist2 = _tc_dist(jnp.concatenate([lab_s[mh:], perm[mh:]]), q, queue,
                     mh, d, s)
    ks2 = sel(dist2)
    rep1 = _tc_extract(jnp.concatenate([lab_s[:mh], ks1]), queue, mh, d, s)
    rep2 = _tc_extract(jnp.concatenate([lab_s[mh:], ks2]), queue, mh, d, s)
    return q.at[perm].set(jnp.concatenate([rep1, rep2]))


# speedup vs baseline: 2.7912x; 1.0086x over previous
import functools

import jax
import jax.numpy as jnp
from jax import lax
from jax.experimental import pallas as pl
from jax.experimental.pallas import tpu as pltpu
from jax.experimental.pallas import tpu_sc as plsc

_K_NN = 5
_W = 8
_L = 16
_LW = 128


def _dist_body(lab_ref, perm_ref, *rest):
    q_refs = rest[:_W]
    queue_refs = rest[_W:2 * _W]
    dist_ref = rest[2 * _W]
    dists = []
    for j in range(_W):
        tile = queue_refs[j][0]
        qv = q_refs[j][0]
        dot = jax.lax.dot_general(
            qv, tile, (((1,), (0,)), ((), ())),
            precision=jax.lax.Precision.HIGHEST,
            preferred_element_type=jnp.float32)
        nrm = jnp.sum(tile * tile, axis=0, keepdims=True)
        inv = 1.0 / (jnp.sqrt(nrm) + 1e-12)
        dists.append(nrm * inv * inv - 2.0 * dot * inv)
    dist_ref[:, 0, 0] = jnp.concatenate(dists, axis=0)


def _tc_dist(lab_s, perm, q, queue, m, d, s):
    rows = m // _W

    def out_map(i, lr, pr):
        return (0, i, 0, 0)

    def q_map(j):
        def f(i, lr, pr):
            return (pr[j * rows + i], 0, 0)
        return f

    def queue_map(j):
        def f(i, lr, pr):
            return (lr[j * rows + i], 0, 0)
        return f

    dist = pl.pallas_call(
        _dist_body,
        grid_spec=pltpu.PrefetchScalarGridSpec(
            num_scalar_prefetch=2,
            grid=(rows,),
            in_specs=[pl.BlockSpec((1, 1, d), q_map(j)) for j in range(_W)] +
                     [pl.BlockSpec((1, d, s), queue_map(j)) for j in range(_W)],
            out_specs=pl.BlockSpec((_W, 1, 1, s), out_map),
        ),
        out_shape=jax.ShapeDtypeStruct((_W, rows, 1, s), jnp.float32),
    )(lab_s, perm, *([q.reshape(-1, 1, d)] * _W), *([queue] * _W))
    return dist.reshape(m, s)


def _bcast16(x, dtype=jnp.int32):
    return jnp.full((_L,), x, dtype)


def _iota16():
    return lax.broadcasted_iota(jnp.int32, (_L,), 0)


def _take16(v, idxvec):
    dnums = lax.GatherDimensionNumbers(
        offset_dims=(), collapsed_slice_dims=(0,), start_index_map=(0,))
    return lax.gather(v, idxvec[:, None], dnums, slice_sizes=(1,),
                      mode=lax.GatherScatterMode.PROMISE_IN_BOUNDS)


def _lanemin16(v):
    for sh in (8, 4, 2, 1):
        v = jnp.minimum(v, _take16(v, _iota16() ^ sh))
    return v


def _make_sc_select(m, s):
    info = plsc.get_sparse_core_info()
    nw = info.num_cores * info.num_subcores
    per_w = m // nw
    sgr = s // _L
    mesh = plsc.VectorSubcoreMesh(core_axis_name="c", subcore_axis_name="s")

    @functools.partial(
        pl.kernel,
        out_type=jax.ShapeDtypeStruct((m,), jnp.int32),
        mesh=mesh,
        scratch_types=[
            pltpu.VMEM((per_w, s), jnp.float32),
            pltpu.VMEM((s,), jnp.float32),
            pltpu.VMEM((per_w,), jnp.int32),
        ],
    )
    def sc_select(dist_hbm, out_hbm, dist_v, row_v, kth_v):
        wid = lax.axis_index("s") * info.num_cores + lax.axis_index("c")
        base = wid * per_w
        pltpu.sync_copy(dist_hbm.at[pl.ds(base, per_w)], dist_v)
        iota = _iota16()
        pos_inf = jnp.float32(jnp.inf)

        def one_group(g, _):
            def one_query(l, acc):
                i = g * _L + l

                def cp_row(k, _):
                    row_v[pl.ds(k * _L, _L)] = dist_v[i, pl.ds(k * _L, _L)]
                    return 0

                lax.fori_loop(0, sgr, cp_row, 0)

                kth_b = _bcast16(0)
                for _r in range(_K_NN):
                    def run_min(k, mv):
                        return jnp.minimum(mv, row_v[pl.ds(k * _L, _L)])

                    mvec = lax.fori_loop(0, sgr, run_min,
                                         jnp.full((_L,), pos_inf, jnp.float32))
                    lo_b = _lanemin16(mvec)

                    def find_min_idx(k, gm):
                        v = row_v[pl.ds(k * _L, _L)]
                        cand = jnp.where(v <= lo_b, k * _L + iota,
                                         jnp.int32(2 ** 30))
                        return jnp.minimum(gm, cand)

                    gmin = lax.fori_loop(0, sgr, find_min_idx,
                                         jnp.full((_L,), 2 ** 30, jnp.int32))
                    kth_b = _lanemin16(gmin)
                    kc = kth_b[0] // _L
                    blk = row_v[pl.ds(kc * _L, _L)]
                    row_v[pl.ds(kc * _L, _L)] = jnp.where(
                        iota == kth_b % _L, pos_inf, blk)

                return jnp.where(iota == l, kth_b, acc)

            picks = lax.fori_loop(0, _L, one_query, jnp.zeros((_L,), jnp.int32))
            kth_v[pl.ds(g * _L, _L)] = picks
            return 0

        lax.fori_loop(0, per_w // _L, one_group, 0)
        pltpu.sync_copy(kth_v, out_hbm.at[pl.ds(base, per_w)])

    return sc_select


def _extract_body(lab_ref, ks_ref, *rest):
    queue_refs = rest[:_W]
    out_ref = rest[_W]
    rows = ks_ref.shape[0] // _W
    i = pl.program_id(0)
    lane = jax.lax.broadcasted_iota(jnp.int32, (1, _LW), 1)
    for j in range(_W):
        win = queue_refs[j][0]
        kth = ks_ref[j * rows + i]
        mask = (lane == kth % _LW).astype(jnp.float32)
        col = jax.lax.dot_general(
            mask, win, (((1,), (1,)), ((), ())),
            preferred_element_type=jnp.float32)
        nrm = jnp.sum(col * col)
        out_ref[j, 0] = col / (jnp.sqrt(nrm) + 1e-12)


def _tc_extract(lab_s, ks, queue, m, d, s):
    rows = m // _W

    def out_map(i, lr, kr):
        return (0, i, 0, 0)

    def queue_map(j):
        def f(i, lr, kr):
            return (lr[j * rows + i], 0, kr[j * rows + i] // _LW)
        return f

    rep = pl.pallas_call(
        _extract_body,
        grid_spec=pltpu.PrefetchScalarGridSpec(
            num_scalar_prefetch=2,
            grid=(rows,),
            in_specs=[pl.BlockSpec((1, d, _LW), queue_map(j))
                      for j in range(_W)],
            out_specs=pl.BlockSpec((_W, 1, 1, d), out_map),
        ),
        out_shape=jax.ShapeDtypeStruct((_W, rows, 1, d), jnp.float32),
    )(lab_s, ks, *([queue] * _W))
    return rep.reshape(m, d)


def kernel(q, labels, queue):
    n, d = q.shape
    c, _, s = queue.shape
    m = n // 2
    lab = labels[:m].astype(jnp.int32)
    packed = jnp.sort(lab * 2048 + jnp.arange(m, dtype=jnp.int32))
    perm = packed & 2047
    lab_s = packed >> 11
    mh = m // 2
    sel = _make_sc_select(mh, s)
    dist1 = _tc_dist(lab_s[:mh], perm[:mh], q, queue, mh, d, s)
    ks1 = sel(dist1)
    dist2 = _tc_dist(lab_s[mh:], perm[mh:], q, queue, mh, d, s)
    ks2 = sel(dist2)
    rep1 = _tc_extract(lab_s[:mh], ks1, queue, mh, d, s)
    rep2 = _tc_extract(lab_s[mh:], ks2, queue, mh, d, s)
    return q.at[perm].set(jnp.concatenate([rep1, rep2]))


# speedup vs baseline: 3.0491x; 1.0924x over previous
import functools

import jax
import jax.numpy as jnp
from jax import lax
from jax.experimental import pallas as pl
from jax.experimental.pallas import tpu as pltpu
from jax.experimental.pallas import tpu_sc as plsc

_K_NN = 5
_W = 16
_L = 16
_LW = 128


def _dist_body(lab_ref, perm_ref, *rest):
    q_refs = rest[:_W]
    queue_refs = rest[_W:2 * _W]
    dist_ref = rest[2 * _W]
    dists = []
    for j in range(_W):
        tile = queue_refs[j][0]
        qv = q_refs[j][0]
        dot = jax.lax.dot_general(
            qv, tile, (((1,), (0,)), ((), ())),
            precision=jax.lax.Precision.HIGHEST,
            preferred_element_type=jnp.float32)
        nrm = jnp.sum(tile * tile, axis=0, keepdims=True)
        inv = 1.0 / (jnp.sqrt(nrm) + 1e-12)
        dists.append(nrm * inv * inv - 2.0 * dot * inv)
    dist_ref[:, 0, 0] = jnp.concatenate(dists, axis=0)


def _tc_dist(lab_s, perm, q, queue, m, d, s):
    rows = m // _W

    def out_map(i, lr, pr):
        return (0, i, 0, 0)

    def q_map(j):
        def f(i, lr, pr):
            return (pr[j * rows + i], 0, 0)
        return f

    def queue_map(j):
        def f(i, lr, pr):
            return (lr[j * rows + i], 0, 0)
        return f

    dist = pl.pallas_call(
        _dist_body,
        grid_spec=pltpu.PrefetchScalarGridSpec(
            num_scalar_prefetch=2,
            grid=(rows,),
            in_specs=[pl.BlockSpec((1, 1, d), q_map(j)) for j in range(_W)] +
                     [pl.BlockSpec((1, d, s), queue_map(j)) for j in range(_W)],
            out_specs=pl.BlockSpec((_W, 1, 1, s), out_map),
        ),
        out_shape=jax.ShapeDtypeStruct((_W, rows, 1, s), jnp.float32),
    )(lab_s, perm, *([q.reshape(-1, 1, d)] * _W), *([queue] * _W))
    return dist.reshape(m, s)


def _bcast16(x, dtype=jnp.int32):
    return jnp.full((_L,), x, dtype)


def _iota16():
    return lax.broadcasted_iota(jnp.int32, (_L,), 0)


def _take16(v, idxvec):
    dnums = lax.GatherDimensionNumbers(
        offset_dims=(), collapsed_slice_dims=(0,), start_index_map=(0,))
    return lax.gather(v, idxvec[:, None], dnums, slice_sizes=(1,),
                      mode=lax.GatherScatterMode.PROMISE_IN_BOUNDS)


def _lanemin16(v):
    for sh in (8, 4, 2, 1):
        v = jnp.minimum(v, _take16(v, _iota16() ^ sh))
    return v


def _make_sc_select(m, s):
    info = plsc.get_sparse_core_info()
    nw = info.num_cores * info.num_subcores
    per_w = m // nw
    sgr = s // _L
    mesh = plsc.VectorSubcoreMesh(core_axis_name="c", subcore_axis_name="s")

    @functools.partial(
        pl.kernel,
        out_type=jax.ShapeDtypeStruct((m,), jnp.int32),
        mesh=mesh,
        scratch_types=[
            pltpu.VMEM((per_w, s), jnp.float32),
            pltpu.VMEM((s,), jnp.float32),
            pltpu.VMEM((per_w,), jnp.int32),
        ],
    )
    def sc_select(dist_hbm, out_hbm, dist_v, row_v, kth_v):
        wid = lax.axis_index("s") * info.num_cores + lax.axis_index("c")
        base = wid * per_w
        pltpu.sync_copy(dist_hbm.at[pl.ds(base, per_w)], dist_v)
        iota = _iota16()
        pos_inf = jnp.float32(jnp.inf)

        def one_group(g, _):
            def one_query(l, acc):
                i = g * _L + l

                def cp_row(k, _):
                    row_v[pl.ds(k * _L, _L)] = dist_v[i, pl.ds(k * _L, _L)]
                    return 0

                lax.fori_loop(0, sgr, cp_row, 0)

                kth_b = _bcast16(0)
                for _r in range(_K_NN):
                    def run_min(k, mv):
                        return jnp.minimum(mv, row_v[pl.ds(k * _L, _L)])

                    mvec = lax.fori_loop(0, sgr, run_min,
                                         jnp.full((_L,), pos_inf, jnp.float32))
                    lo_b = _lanemin16(mvec)

                    def find_min_idx(k, gm):
                        v = row_v[pl.ds(k * _L, _L)]
                        cand = jnp.where(v <= lo_b, k * _L + iota,
                                         jnp.int32(2 ** 30))
                        return jnp.minimum(gm, cand)

                    gmin = lax.fori_loop(0, sgr, find_min_idx,
                                         jnp.full((_L,), 2 ** 30, jnp.int32))
                    kth_b = _lanemin16(gmin)
                    kc = kth_b[0] // _L
                    blk = row_v[pl.ds(kc * _L, _L)]
                    row_v[pl.ds(kc * _L, _L)] = jnp.where(
                        iota == kth_b % _L, pos_inf, blk)

                return jnp.where(iota == l, kth_b, acc)

            picks = lax.fori_loop(0, _L, one_query, jnp.zeros((_L,), jnp.int32))
            kth_v[pl.ds(g * _L, _L)] = picks
            return 0

        lax.fori_loop(0, per_w // _L, one_group, 0)
        pltpu.sync_copy(kth_v, out_hbm.at[pl.ds(base, per_w)])

    return sc_select


def _extract_body(lab_ref, ks_ref, *rest):
    queue_refs = rest[:_W]
    out_ref = rest[_W]
    rows = ks_ref.shape[0] // _W
    i = pl.program_id(0)
    lane = jax.lax.broadcasted_iota(jnp.int32, (1, _LW), 1)
    for j in range(_W):
        win = queue_refs[j][0]
        kth = ks_ref[j * rows + i]
        mask = (lane == kth % _LW).astype(jnp.float32)
        col = jax.lax.dot_general(
            mask, win, (((1,), (1,)), ((), ())),
            preferred_element_type=jnp.float32)
        nrm = jnp.sum(col * col)
        out_ref[j, 0] = col / (jnp.sqrt(nrm) + 1e-12)


def _tc_extract(lab_s, ks, queue, m, d, s):
    rows = m // _W

    def out_map(i, lr, kr):
        return (0, i, 0, 0)

    def queue_map(j):
        def f(i, lr, kr):
            return (lr[j * rows + i], 0, kr[j * rows + i] // _LW)
        return f

    rep = pl.pallas_call(
        _extract_body,
        grid_spec=pltpu.PrefetchScalarGridSpec(
            num_scalar_prefetch=2,
            grid=(rows,),
            in_specs=[pl.BlockSpec((1, d, _LW), queue_map(j))
                      for j in range(_W)],
            out_specs=pl.BlockSpec((_W, 1, 1, d), out_map),
        ),
        out_shape=jax.ShapeDtypeStruct((_W, rows, 1, d), jnp.float32),
    )(lab_s, ks, *([queue] * _W))
    return rep.reshape(m, d)


def kernel(q, labels, queue):
    n, d = q.shape
    c, _, s = queue.shape
    m = n // 2
    lab = labels[:m].astype(jnp.int32)
    packed = jnp.sort(lab * 2048 + jnp.arange(m, dtype=jnp.int32))
    perm = packed & 2047
    lab_s = packed >> 11
    mh = m // 2
    sel = _make_sc_select(mh, s)
    dist1 = _tc_dist(lab_s[:mh], perm[:mh], q, queue, mh, d, s)
    ks1 = sel(dist1)
    dist2 = _tc_dist(lab_s[mh:], perm[mh:], q, queue, mh, d, s)
    ks2 = sel(dist2)
    rep1 = _tc_extract(lab_s[:mh], ks1, queue, mh, d, s)
    rep2 = _tc_extract(lab_s[mh:], ks2, queue, mh, d, s)
    return q.at[perm].set(jnp.concatenate([rep1, rep2]))


# speedup vs baseline: 3.1103x; 1.0201x over previous
import functools

import jax
import jax.numpy as jnp
from jax import lax
from jax.experimental import pallas as pl
from jax.experimental.pallas import tpu as pltpu
from jax.experimental.pallas import tpu_sc as plsc

_K_NN = 5
_W = 32
_L = 16
_LW = 128


def _dist_body(lab_ref, perm_ref, *rest):
    q_refs = rest[:_W]
    queue_refs = rest[_W:2 * _W]
    dist_ref = rest[2 * _W]
    dists = []
    for j in range(_W):
        tile = queue_refs[j][0]
        qv = q_refs[j][0]
        dot = jax.lax.dot_general(
            qv, tile, (((1,), (0,)), ((), ())),
            precision=jax.lax.Precision.HIGHEST,
            preferred_element_type=jnp.float32)
        nrm = jnp.sum(tile * tile, axis=0, keepdims=True)
        inv = 1.0 / (jnp.sqrt(nrm) + 1e-12)
        dists.append(nrm * inv * inv - 2.0 * dot * inv)
    dist_ref[:, 0, 0] = jnp.concatenate(dists, axis=0)


def _tc_dist(lab_s, perm, q, queue, m, d, s):
    rows = m // _W

    def out_map(i, lr, pr):
        return (0, i, 0, 0)

    def q_map(j):
        def f(i, lr, pr):
            return (pr[j * rows + i], 0, 0)
        return f

    def queue_map(j):
        def f(i, lr, pr):
            return (lr[j * rows + i], 0, 0)
        return f

    dist = pl.pallas_call(
        _dist_body,
        grid_spec=pltpu.PrefetchScalarGridSpec(
            num_scalar_prefetch=2,
            grid=(rows,),
            in_specs=[pl.BlockSpec((1, 1, d), q_map(j)) for j in range(_W)] +
                     [pl.BlockSpec((1, d, s), queue_map(j)) for j in range(_W)],
            out_specs=pl.BlockSpec((_W, 1, 1, s), out_map),
        ),
        out_shape=jax.ShapeDtypeStruct((_W, rows, 1, s), jnp.float32),
    )(lab_s, perm, *([q.reshape(-1, 1, d)] * _W), *([queue] * _W))
    return dist.reshape(m, s)


def _bcast16(x, dtype=jnp.int32):
    return jnp.full((_L,), x, dtype)


def _iota16():
    return lax.broadcasted_iota(jnp.int32, (_L,), 0)


def _take16(v, idxvec):
    dnums = lax.GatherDimensionNumbers(
        offset_dims=(), collapsed_slice_dims=(0,), start_index_map=(0,))
    return lax.gather(v, idxvec[:, None], dnums, slice_sizes=(1,),
                      mode=lax.GatherScatterMode.PROMISE_IN_BOUNDS)


def _lanemin16(v):
    for sh in (8, 4, 2, 1):
        v = jnp.minimum(v, _take16(v, _iota16() ^ sh))
    return v


def _make_sc_select(m, s):
    info = plsc.get_sparse_core_info()
    nw = info.num_cores * info.num_subcores
    per_w = m // nw
    sgr = s // _L
    mesh = plsc.VectorSubcoreMesh(core_axis_name="c", subcore_axis_name="s")

    @functools.partial(
        pl.kernel,
        out_type=jax.ShapeDtypeStruct((m,), jnp.int32),
        mesh=mesh,
        scratch_types=[
            pltpu.VMEM((per_w, s), jnp.float32),
            pltpu.VMEM((s,), jnp.float32),
            pltpu.VMEM((per_w,), jnp.int32),
        ],
    )
    def sc_select(dist_hbm, out_hbm, dist_v, row_v, kth_v):
        wid = lax.axis_index("s") * info.num_cores + lax.axis_index("c")
        base = wid * per_w
        pltpu.sync_copy(dist_hbm.at[pl.ds(base, per_w)], dist_v)
        iota = _iota16()
        pos_inf = jnp.float32(jnp.inf)

        def one_group(g, _):
            def one_query(l, acc):
                i = g * _L + l

                def cp_row(k, _):
                    row_v[pl.ds(k * _L, _L)] = dist_v[i, pl.ds(k * _L, _L)]
                    return 0

                lax.fori_loop(0, sgr, cp_row, 0)

                kth_b = _bcast16(0)
                for _r in range(_K_NN):
                    def run_min(k, mv):
                        return jnp.minimum(mv, row_v[pl.ds(k * _L, _L)])

                    mvec = lax.fori_loop(0, sgr, run_min,
                                         jnp.full((_L,), pos_inf, jnp.float32))
                    lo_b = _lanemin16(mvec)

                    def find_min_idx(k, gm):
                        v = row_v[pl.ds(k * _L, _L)]
                        cand = jnp.where(v <= lo_b, k * _L + iota,
                                         jnp.int32(2 ** 30))
                        return jnp.minimum(gm, cand)

                    gmin = lax.fori_loop(0, sgr, find_min_idx,
                                         jnp.full((_L,), 2 ** 30, jnp.int32))
                    kth_b = _lanemin16(gmin)
                    kc = kth_b[0] // _L
                    blk = row_v[pl.ds(kc * _L, _L)]
                    row_v[pl.ds(kc * _L, _L)] = jnp.where(
                        iota == kth_b % _L, pos_inf, blk)

                return jnp.where(iota == l, kth_b, acc)

            picks = lax.fori_loop(0, _L, one_query, jnp.zeros((_L,), jnp.int32))
            kth_v[pl.ds(g * _L, _L)] = picks
            return 0

        lax.fori_loop(0, per_w // _L, one_group, 0)
        pltpu.sync_copy(kth_v, out_hbm.at[pl.ds(base, per_w)])

    return sc_select


def _extract_body(lab_ref, ks_ref, *rest):
    queue_refs = rest[:_W]
    out_ref = rest[_W]
    rows = ks_ref.shape[0] // _W
    i = pl.program_id(0)
    lane = jax.lax.broadcasted_iota(jnp.int32, (1, _LW), 1)
    for j in range(_W):
        win = queue_refs[j][0]
        kth = ks_ref[j * rows + i]
        mask = (lane == kth % _LW).astype(jnp.float32)
        col = jax.lax.dot_general(
            mask, win, (((1,), (1,)), ((), ())),
            preferred_element_type=jnp.float32)
        nrm = jnp.sum(col * col)
        out_ref[j, 0] = col / (jnp.sqrt(nrm) + 1e-12)


def _tc_extract(lab_s, ks, queue, m, d, s):
    rows = m // _W

    def out_map(i, lr, kr):
        return (0, i, 0, 0)

    def queue_map(j):
        def f(i, lr, kr):
            return (lr[j * rows + i], 0, kr[j * rows + i] // _LW)
        return f

    rep = pl.pallas_call(
        _extract_body,
        grid_spec=pltpu.PrefetchScalarGridSpec(
            num_scalar_prefetch=2,
            grid=(rows,),
            in_specs=[pl.BlockSpec((1, d, _LW), queue_map(j))
                      for j in range(_W)],
            out_specs=pl.BlockSpec((_W, 1, 1, d), out_map),
        ),
        out_shape=jax.ShapeDtypeStruct((_W, rows, 1, d), jnp.float32),
    )(lab_s, ks, *([queue] * _W))
    return rep.reshape(m, d)


def kernel(q, labels, queue):
    n, d = q.shape
    c, _, s = queue.shape
    m = n // 2
    lab = labels[:m].astype(jnp.int32)
    packed = jnp.sort(lab * 2048 + jnp.arange(m, dtype=jnp.int32))
    perm = packed & 2047
    lab_s = packed >> 11
    mh = m // 2
    sel = _make_sc_select(mh, s)
    dist1 = _tc_dist(lab_s[:mh], perm[:mh], q, queue, mh, d, s)
    ks1 = sel(dist1)
    dist2 = _tc_dist(lab_s[mh:], perm[mh:], q, queue, mh, d, s)
    ks2 = sel(dist2)
    rep1 = _tc_extract(lab_s[:mh], ks1, queue, mh, d, s)
    rep2 = _tc_extract(lab_s[mh:], ks2, queue, mh, d, s)
    return q.at[perm].set(jnp.concatenate([rep1, rep2]))


# speedup vs baseline: 4.9381x; 1.5876x over previous
import functools

import jax
import jax.numpy as jnp
from jax import lax
from jax.experimental import pallas as pl
from jax.experimental.pallas import tpu as pltpu
from jax.experimental.pallas import tpu_sc as plsc

_K_NN = 5
_W = 32
_L = 16
_LW = 128


def _dist_body(lab_ref, perm_ref, *rest):
    q_refs = rest[:_W]
    queue_refs = rest[_W:2 * _W]
    dist_ref = rest[2 * _W]
    dists = []
    for j in range(_W):
        tile = queue_refs[j][0]
        qv = q_refs[j][0]
        qcol = qv.reshape(tile.shape[0], 1)
        dot = jnp.sum(tile * qcol, axis=0, keepdims=True)
        nrm = jnp.sum(tile * tile, axis=0, keepdims=True)
        inv = 1.0 / (jnp.sqrt(nrm) + 1e-12)
        dists.append(nrm * inv * inv - 2.0 * dot * inv)
    dist_ref[:, 0, 0] = jnp.concatenate(dists, axis=0)


def _tc_dist(lab_s, perm, q, queue, m, d, s):
    rows = m // _W

    def out_map(i, lr, pr):
        return (0, i, 0, 0)

    def q_map(j):
        def f(i, lr, pr):
            return (pr[j * rows + i], 0, 0)
        return f

    def queue_map(j):
        def f(i, lr, pr):
            return (lr[j * rows + i], 0, 0)
        return f

    dist = pl.pallas_call(
        _dist_body,
        grid_spec=pltpu.PrefetchScalarGridSpec(
            num_scalar_prefetch=2,
            grid=(rows,),
            in_specs=[pl.BlockSpec((1, 1, d), q_map(j)) for j in range(_W)] +
                     [pl.BlockSpec((1, d, s), queue_map(j)) for j in range(_W)],
            out_specs=pl.BlockSpec((_W, 1, 1, s), out_map),
        ),
        out_shape=jax.ShapeDtypeStruct((_W, rows, 1, s), jnp.float32),
    )(lab_s, perm, *([q.reshape(-1, 1, d)] * _W), *([queue] * _W))
    return dist.reshape(m, s)


def _bcast16(x, dtype=jnp.int32):
    return jnp.full((_L,), x, dtype)


def _iota16():
    return lax.broadcasted_iota(jnp.int32, (_L,), 0)


def _take16(v, idxvec):
    dnums = lax.GatherDimensionNumbers(
        offset_dims=(), collapsed_slice_dims=(0,), start_index_map=(0,))
    return lax.gather(v, idxvec[:, None], dnums, slice_sizes=(1,),
                      mode=lax.GatherScatterMode.PROMISE_IN_BOUNDS)


def _lanemin16(v):
    for sh in (8, 4, 2, 1):
        v = jnp.minimum(v, _take16(v, _iota16() ^ sh))
    return v


def _make_sc_select(m, s):
    info = plsc.get_sparse_core_info()
    nw = info.num_cores * info.num_subcores
    per_w = m // nw
    sgr = s // _L
    mesh = plsc.VectorSubcoreMesh(core_axis_name="c", subcore_axis_name="s")

    @functools.partial(
        pl.kernel,
        out_type=jax.ShapeDtypeStruct((m,), jnp.int32),
        mesh=mesh,
        scratch_types=[
            pltpu.VMEM((per_w, s), jnp.float32),
            pltpu.VMEM((s,), jnp.float32),
            pltpu.VMEM((per_w,), jnp.int32),
        ],
    )
    def sc_select(dist_hbm, out_hbm, dist_v, row_v, kth_v):
        wid = lax.axis_index("s") * info.num_cores + lax.axis_index("c")
        base = wid * per_w
        pltpu.sync_copy(dist_hbm.at[pl.ds(base, per_w)], dist_v)
        iota = _iota16()
        pos_inf = jnp.float32(jnp.inf)

        def one_group(g, _):
            def one_query(l, acc):
                i = g * _L + l

                def cp_row(k, _):
                    row_v[pl.ds(k * _L, _L)] = dist_v[i, pl.ds(k * _L, _L)]
                    return 0

                lax.fori_loop(0, sgr, cp_row, 0)

                kth_b = _bcast16(0)
                for _r in range(_K_NN):
                    def run_min(k, mv):
                        return jnp.minimum(mv, row_v[pl.ds(k * _L, _L)])

                    mvec = lax.fori_loop(0, sgr, run_min,
                                         jnp.full((_L,), pos_inf, jnp.float32))
                    lo_b = _lanemin16(mvec)

                    def find_min_idx(k, gm):
                        v = row_v[pl.ds(k * _L, _L)]
                        cand = jnp.where(v <= lo_b, k * _L + iota,
                                         jnp.int32(2 ** 30))
                        return jnp.minimum(gm, cand)

                    gmin = lax.fori_loop(0, sgr, find_min_idx,
                                         jnp.full((_L,), 2 ** 30, jnp.int32))
                    kth_b = _lanemin16(gmin)
                    kc = kth_b[0] // _L
                    blk = row_v[pl.ds(kc * _L, _L)]
                    row_v[pl.ds(kc * _L, _L)] = jnp.where(
                        iota == kth_b % _L, pos_inf, blk)

                return jnp.where(iota == l, kth_b, acc)

            picks = lax.fori_loop(0, _L, one_query, jnp.zeros((_L,), jnp.int32))
            kth_v[pl.ds(g * _L, _L)] = picks
            return 0

        lax.fori_loop(0, per_w // _L, one_group, 0)
        pltpu.sync_copy(kth_v, out_hbm.at[pl.ds(base, per_w)])

    return sc_select


def _extract_body(lab_ref, ks_ref, *rest):
    queue_refs = rest[:_W]
    out_ref = rest[_W]
    rows = ks_ref.shape[0] // _W
    i = pl.program_id(0)
    lane = jax.lax.broadcasted_iota(jnp.int32, (1, _LW), 1)
    for j in range(_W):
        win = queue_refs[j][0]
        kth = ks_ref[j * rows + i]
        mask = (lane == kth % _LW).astype(jnp.float32)
        col = jax.lax.dot_general(
            mask, win, (((1,), (1,)), ((), ())),
            preferred_element_type=jnp.float32)
        nrm = jnp.sum(col * col)
        out_ref[j, 0] = col / (jnp.sqrt(nrm) + 1e-12)


def _tc_extract(lab_s, ks, queue, m, d, s):
    rows = m // _W

    def out_map(i, lr, kr):
        return (0, i, 0, 0)

    def queue_map(j):
        def f(i, lr, kr):
            return (lr[j * rows + i], 0, kr[j * rows + i] // _LW)
        return f

    rep = pl.pallas_call(
        _extract_body,
        grid_spec=pltpu.PrefetchScalarGridSpec(
            num_scalar_prefetch=2,
            grid=(rows,),
            in_specs=[pl.BlockSpec((1, d, _LW), queue_map(j))
                      for j in range(_W)],
            out_specs=pl.BlockSpec((_W, 1, 1, d), out_map),
        ),
        out_shape=jax.ShapeDtypeStruct((_W, rows, 1, d), jnp.float32),
    )(lab_s, ks, *([queue] * _W))
    return rep.reshape(m, d)


def kernel(q, labels, queue):
    n, d = q.shape
    c, _, s = queue.shape
    m = n // 2
    lab = labels[:m].astype(jnp.int32)
    packed = jnp.sort(lab * 2048 + jnp.arange(m, dtype=jnp.int32))
    perm = packed & 2047
    lab_s = packed >> 11
    mh = m // 2
    sel = _make_sc_select(mh, s)
    dist1 = _tc_dist(lab_s[:mh], perm[:mh], q, queue, mh, d, s)
    ks1 = sel(dist1)
    dist2 = _tc_dist(lab_s[mh:], perm[mh:], q, queue, mh, d, s)
    ks2 = sel(dist2)
    rep1 = _tc_extract(lab_s[:mh], ks1, queue, mh, d, s)
    rep2 = _tc_extract(lab_s[mh:], ks2, queue, mh, d, s)
    return q.at[perm].set(jnp.concatenate([rep1, rep2]))
